# Initial kernel scaffold; baseline (speedup 1.0000x reference)
import jax
import jax.numpy as jnp
from jax.experimental import pallas as pl


def kernel(t, week_emb, day_emb, month_emb, hour_emb):
    raise NotImplementedError("write your pallas kernel here")


# speedup vs baseline: 1.1908x; 1.1908x over previous
import functools

import jax
import jax.numpy as jnp
from jax import lax
from jax.experimental import pallas as pl
from jax.experimental.pallas import tpu as pltpu
from jax.experimental.pallas import tpu_sc as plsc

DIM = 128
BATCH = 16384
LANES = 16
CHUNK = 128


def _hour_from_unix(tv):
    n = lax.shift_right_logical(tv, 7)
    q = (n.astype(jnp.float32) * jnp.float32(1.0 / 675.0)).astype(jnp.int32)
    r = tv - q * 86400
    r = jnp.where(r < 0, r + 86400, r)
    r = jnp.where(r >= 86400, r - 86400, r)
    h = (r.astype(jnp.float32) * jnp.float32(1.0 / 3600.0)).astype(jnp.int32)
    rem = r - h * 3600
    h = jnp.where(rem < 0, h - 1, h)
    rem = jnp.where(rem < 0, rem + 3600, rem)
    h = jnp.where(rem >= 3600, h + 1, h)
    return h


def kernel(t, week_emb, day_emb, month_emb, hour_emb):
    del week_emb, day_emb, month_emb
    info = plsc.get_sparse_core_info()
    nc, ns = info.num_cores, info.num_subcores
    nw = nc * ns
    bpw = BATCH // nw
    nchunks = bpw // CHUNK

    mesh = plsc.VectorSubcoreMesh(core_axis_name="c", subcore_axis_name="s")

    @functools.partial(
        pl.kernel,
        mesh=mesh,
        out_type=jax.ShapeDtypeStruct((BATCH, DIM), jnp.float32),
        scratch_types=[
            pltpu.VMEM((bpw,), jnp.int32),
            pltpu.VMEM((nchunks, CHUNK), jnp.int32),
            pltpu.VMEM((nchunks, CHUNK, DIM), jnp.float32),
            pltpu.SemaphoreType.DMA,
            pltpu.SemaphoreType.DMA,
            pltpu.SemaphoreType.DMA,
            pltpu.SemaphoreType.DMA,
            pltpu.SemaphoreType.DMA,
        ],
    )
    def sc_lookup(t_hbm, tab_hbm, out_hbm, t_v, idx_v, rows_v,
                  g0, g1, g2, g3, ssem):
        gsems = [g0, g1, g2, g3]
        wid = lax.axis_index("s") * nc + lax.axis_index("c")
        base = wid * bpw

        pltpu.sync_copy(t_hbm.at[pl.ds(base, bpw)], t_v)

        for i in range(bpw // LANES):
            tv = t_v[pl.ds(i * LANES, LANES)]
            idx_v[i // (CHUNK // LANES),
                  pl.ds((i % (CHUNK // LANES)) * LANES, LANES)] = _hour_from_unix(tv)

        gathers = []
        for j in range(nchunks):
            gathers.append(
                pltpu.async_copy(tab_hbm.at[idx_v.at[j]], rows_v.at[j], gsems[j]))
        scatters = []
        for j in range(nchunks):
            gathers[j].wait()
            scatters.append(
                pltpu.async_copy(rows_v.at[j],
                                 out_hbm.at[pl.ds(base + j * CHUNK, CHUNK)], ssem))
        for j in range(nchunks):
            scatters[j].wait()

    return sc_lookup(t, hour_emb)


# speedup vs baseline: 2.7025x; 2.2694x over previous
import functools

import jax
import jax.numpy as jnp
from jax import lax
from jax.experimental import pallas as pl
from jax.experimental.pallas import tpu as pltpu
from jax.experimental.pallas import tpu_sc as plsc

DIM = 128
BATCH = 16384
LANES = 16
CHUNK = 128


def _hour_from_unix(tv):
    n = lax.shift_right_logical(tv, 7)
    q = (n.astype(jnp.float32) * jnp.float32(1.0 / 675.0)).astype(jnp.int32)
    r = tv - q * 86400
    r = jnp.where(r < 0, r + 86400, r)
    r = jnp.where(r >= 86400, r - 86400, r)
    h = (r.astype(jnp.float32) * jnp.float32(1.0 / 3600.0)).astype(jnp.int32)
    rem = r - h * 3600
    h = jnp.where(rem < 0, h - 1, h)
    rem = jnp.where(rem < 0, rem + 3600, rem)
    h = jnp.where(rem >= 3600, h + 1, h)
    return h


def kernel(t, week_emb, day_emb, month_emb, hour_emb):
    del week_emb, day_emb, month_emb
    info = plsc.get_sparse_core_info()
    nc, ns = info.num_cores, info.num_subcores
    nw = nc * ns
    bpw = BATCH // nw
    nchunks = bpw // CHUNK

    mesh = plsc.VectorSubcoreMesh(core_axis_name="c", subcore_axis_name="s")

    @functools.partial(
        pl.kernel,
        mesh=mesh,
        out_type=jax.ShapeDtypeStruct((BATCH, DIM), jnp.float32),
        scratch_types=[
            pltpu.VMEM((bpw,), jnp.int32),
            pltpu.VMEM((nchunks, CHUNK), jnp.int32),
            pltpu.VMEM((nchunks, CHUNK, DIM), jnp.float32),
            pltpu.VMEM_SHARED((24, DIM), jnp.float32),
            pltpu.SemaphoreType.DMA,
            pltpu.SemaphoreType.DMA,
            pltpu.SemaphoreType.DMA,
            pltpu.SemaphoreType.DMA,
            pltpu.SemaphoreType.DMA,
        ],
    )
    def sc_lookup(t_hbm, tab_hbm, out_hbm, t_v, idx_v, rows_v, tab_sh,
                  g0, g1, g2, g3, ssem):
        gsems = [g0, g1, g2, g3]
        sid = lax.axis_index("s")
        wid = sid * nc + lax.axis_index("c")
        base = wid * bpw

        @pl.when(sid == 0)
        def _stage_table():
            pltpu.sync_copy(tab_hbm, tab_sh)

        pltpu.sync_copy(t_hbm.at[pl.ds(base, bpw)], t_v)

        for i in range(bpw // LANES):
            tv = t_v[pl.ds(i * LANES, LANES)]
            idx_v[i // (CHUNK // LANES),
                  pl.ds((i % (CHUNK // LANES)) * LANES, LANES)] = _hour_from_unix(tv)

        plsc.subcore_barrier()

        gathers = []
        for j in range(nchunks):
            gathers.append(
                pltpu.async_copy(tab_sh.at[idx_v.at[j]], rows_v.at[j], gsems[j]))
        scatters = []
        for j in range(nchunks):
            gathers[j].wait()
            scatters.append(
                pltpu.async_copy(rows_v.at[j],
                                 out_hbm.at[pl.ds(base + j * CHUNK, CHUNK)], ssem))
        for j in range(nchunks):
            scatters[j].wait()

    return sc_lookup(t, hour_emb)


# speedup vs baseline: 2.7193x; 1.0062x over previous
import functools

import jax
import jax.numpy as jnp
from jax import lax
from jax.experimental import pallas as pl
from jax.experimental.pallas import tpu as pltpu
from jax.experimental.pallas import tpu_sc as plsc

DIM = 128
BATCH = 16384
LANES = 16
CHUNK = 128


def _hour_from_unix(tv):
    n = lax.shift_right_logical(tv, 7)
    q = (n.astype(jnp.float32) * jnp.float32(1.0 / 675.0)).astype(jnp.int32)
    r = tv - q * 86400
    r = jnp.where(r < 0, r + 86400, r)
    r = jnp.where(r >= 86400, r - 86400, r)
    h = (r.astype(jnp.float32) * jnp.float32(1.0 / 3600.0)).astype(jnp.int32)
    rem = r - h * 3600
    h = jnp.where(rem < 0, h - 1, h)
    rem = jnp.where(rem < 0, rem + 3600, rem)
    h = jnp.where(rem >= 3600, h + 1, h)
    return h


def kernel(t, week_emb, day_emb, month_emb, hour_emb):
    del week_emb, day_emb, month_emb
    info = plsc.get_sparse_core_info()
    nc, ns = info.num_cores, info.num_subcores
    nw = nc * ns
    bpw = BATCH // nw
    nchunks = bpw // CHUNK

    mesh = plsc.VectorSubcoreMesh(core_axis_name="c", subcore_axis_name="s")

    @functools.partial(
        pl.kernel,
        mesh=mesh,
        out_type=jax.ShapeDtypeStruct((BATCH, DIM), jnp.float32),
        scratch_types=[
            pltpu.VMEM((bpw,), jnp.int32),
            pltpu.VMEM((nchunks, CHUNK), jnp.int32),
            pltpu.VMEM((nchunks, CHUNK, DIM), jnp.float32),
            pltpu.VMEM_SHARED((24, DIM), jnp.float32),
            pltpu.SemaphoreType.DMA,
            pltpu.SemaphoreType.DMA,
            pltpu.SemaphoreType.DMA,
            pltpu.SemaphoreType.DMA,
            pltpu.SemaphoreType.DMA,
        ],
    )
    def sc_lookup(t_hbm, tab_hbm, out_hbm, t_v, idx_v, rows_v, tab_sh,
                  g0, g1, g2, g3, ssem):
        gsems = [g0, g1, g2, g3]
        sid = lax.axis_index("s")
        wid = sid * nc + lax.axis_index("c")
        base = wid * bpw

        @pl.when(sid == 0)
        def _stage_table():
            pltpu.sync_copy(tab_hbm, tab_sh)

        pltpu.sync_copy(t_hbm.at[pl.ds(base, bpw)], t_v)

        def _compute(i, carry):
            row = lax.shift_right_logical(i, 3)
            col = lax.mul(lax.rem(i, 8), LANES)
            tv = t_v[pl.ds(i * LANES, LANES)]
            idx_v[row, pl.ds(col, LANES)] = _hour_from_unix(tv)
            return carry

        lax.fori_loop(0, bpw // LANES, _compute, 0)

        plsc.subcore_barrier()

        gathers = []
        for j in range(nchunks):
            gathers.append(
                pltpu.async_copy(tab_sh.at[idx_v.at[j]], rows_v.at[j], gsems[j]))
        scatters = []
        for j in range(nchunks):
            gathers[j].wait()
            scatters.append(
                pltpu.async_copy(rows_v.at[j],
                                 out_hbm.at[pl.ds(base + j * CHUNK, CHUNK)], ssem))
        for j in range(nchunks):
            scatters[j].wait()

    return sc_lookup(t, hour_emb)


# speedup vs baseline: 2.7213x; 1.0007x over previous
import functools

import jax
import jax.numpy as jnp
from jax import lax
from jax.experimental import pallas as pl
from jax.experimental.pallas import tpu as pltpu
from jax.experimental.pallas import tpu_sc as plsc

DIM = 128
BATCH = 16384
LANES = 16
CHUNK = 128


def _hour_from_unix(tv):
    n = lax.shift_right_logical(tv, 7)
    q = (n.astype(jnp.float32) * jnp.float32(1.0 / 675.0)).astype(jnp.int32)
    r = tv - q * 86400
    r = jnp.where(r < 0, r + 86400, r)
    r = jnp.where(r >= 86400, r - 86400, r)
    h = (r.astype(jnp.float32) * jnp.float32(1.0 / 3600.0)).astype(jnp.int32)
    rem = r - h * 3600
    h = jnp.where(rem < 0, h - 1, h)
    rem = jnp.where(rem < 0, rem + 3600, rem)
    h = jnp.where(rem >= 3600, h + 1, h)
    return h


def kernel(t, week_emb, day_emb, month_emb, hour_emb):
    del week_emb, day_emb, month_emb
    info = plsc.get_sparse_core_info()
    nc, ns = info.num_cores, info.num_subcores
    nw = nc * ns
    bpw = BATCH // nw
    nchunks = bpw // CHUNK

    mesh = plsc.VectorSubcoreMesh(core_axis_name="c", subcore_axis_name="s")

    @functools.partial(
        pl.kernel,
        mesh=mesh,
        out_type=jax.ShapeDtypeStruct((BATCH, DIM), jnp.float32),
        scratch_types=[
            pltpu.VMEM((bpw,), jnp.int32),
            pltpu.VMEM((nchunks, CHUNK), jnp.int32),
            pltpu.VMEM((nchunks, CHUNK, DIM), jnp.float32),
            pltpu.VMEM_SHARED((24, DIM), jnp.float32),
            pltpu.SemaphoreType.DMA,
            pltpu.SemaphoreType.DMA,
            pltpu.SemaphoreType.DMA,
            pltpu.SemaphoreType.DMA,
            pltpu.SemaphoreType.DMA,
        ],
    )
    def sc_lookup(t_hbm, tab_hbm, out_hbm, t_v, idx_v, rows_v, tab_sh,
                  g0, g1, g2, g3, ssem):
        gsems = [g0, g1, g2, g3]
        sid = lax.axis_index("s")
        wid = sid * nc + lax.axis_index("c")
        base = wid * bpw

        @pl.when(sid == 0)
        def _stage_table():
            pltpu.sync_copy(tab_hbm, tab_sh)

        pltpu.sync_copy(t_hbm.at[pl.ds(base, bpw)], t_v)
        plsc.subcore_barrier()

        gathers = []
        for j in range(nchunks):
            def _compute(k, carry, j=j):
                tv = t_v[pl.ds(j * CHUNK + k * LANES, LANES)]
                idx_v[j, pl.ds(k * LANES, LANES)] = _hour_from_unix(tv)
                return carry

            lax.fori_loop(0, CHUNK // LANES, _compute, 0)
            gathers.append(
                pltpu.async_copy(tab_sh.at[idx_v.at[j]], rows_v.at[j], gsems[j]))

        scatters = []
        for j in range(nchunks):
            gathers[j].wait()
            scatters.append(
                pltpu.async_copy(rows_v.at[j],
                                 out_hbm.at[pl.ds(base + j * CHUNK, CHUNK)], ssem))
        for j in range(nchunks):
            scatters[j].wait()

    return sc_lookup(t, hour_emb)


# speedup vs baseline: 2.7381x; 1.0062x over previous
import functools

import jax
import jax.numpy as jnp
from jax import lax
from jax.experimental import pallas as pl
from jax.experimental.pallas import tpu as pltpu
from jax.experimental.pallas import tpu_sc as plsc

DIM = 128
BATCH = 16384
LANES = 16
CHUNK = 64


def _hour_from_unix(tv):
    n = lax.shift_right_logical(tv, 7)
    q = (n.astype(jnp.float32) * jnp.float32(1.0 / 675.0)).astype(jnp.int32)
    r = tv - q * 86400
    r = jnp.where(r < 0, r + 86400, r)
    r = jnp.where(r >= 86400, r - 86400, r)
    h = (r.astype(jnp.float32) * jnp.float32(1.0 / 3600.0)).astype(jnp.int32)
    rem = r - h * 3600
    h = jnp.where(rem < 0, h - 1, h)
    rem = jnp.where(rem < 0, rem + 3600, rem)
    h = jnp.where(rem >= 3600, h + 1, h)
    return h


def kernel(t, week_emb, day_emb, month_emb, hour_emb):
    del week_emb, day_emb, month_emb
    info = plsc.get_sparse_core_info()
    nc, ns = info.num_cores, info.num_subcores
    nw = nc * ns
    bpw = BATCH // nw
    nchunks = bpw // CHUNK

    mesh = plsc.VectorSubcoreMesh(core_axis_name="c", subcore_axis_name="s")

    @functools.partial(
        pl.kernel,
        mesh=mesh,
        out_type=jax.ShapeDtypeStruct((BATCH, DIM), jnp.float32),
        scratch_types=[
            pltpu.VMEM((bpw,), jnp.int32),
            pltpu.VMEM((nchunks, CHUNK), jnp.int32),
            pltpu.VMEM((nchunks, CHUNK, DIM), jnp.float32),
            pltpu.VMEM_SHARED((24, DIM), jnp.float32),
        ] + [pltpu.SemaphoreType.DMA] * (nchunks + 1),
    )
    def sc_lookup(t_hbm, tab_hbm, out_hbm, t_v, idx_v, rows_v, tab_sh, *sems):
        gsems, ssem = sems[:nchunks], sems[nchunks]
        sid = lax.axis_index("s")
        wid = sid * nc + lax.axis_index("c")
        base = wid * bpw

        @pl.when(sid == 0)
        def _stage_table():
            pltpu.sync_copy(tab_hbm, tab_sh)

        pltpu.sync_copy(t_hbm.at[pl.ds(base, bpw)], t_v)
        plsc.subcore_barrier()

        gathers = []
        for j in range(nchunks):
            def _compute(k, carry, j=j):
                tv = t_v[pl.ds(j * CHUNK + k * LANES, LANES)]
                idx_v[j, pl.ds(k * LANES, LANES)] = _hour_from_unix(tv)
                return carry

            lax.fori_loop(0, CHUNK // LANES, _compute, 0)
            gathers.append(
                pltpu.async_copy(tab_sh.at[idx_v.at[j]], rows_v.at[j], gsems[j]))

        scatters = []
        for j in range(nchunks):
            gathers[j].wait()
            scatters.append(
                pltpu.async_copy(rows_v.at[j],
                                 out_hbm.at[pl.ds(base + j * CHUNK, CHUNK)], ssem))
        for j in range(nchunks):
            scatters[j].wait()

    return sc_lookup(t, hour_emb)
